# baseline (speedup 1.0000x reference)
import jax
import jax.numpy as jnp
from jax import lax
from jax.experimental import pallas as pl
from jax.experimental.pallas import tpu as pltpu
from jax.experimental.pallas import tpu_sc as plsc

L = 16
NW = 32
NC = 2
NQ = 4096
NKEY = 100000
NCHUNK = 5
KTOP = 16
PREFIX_V = 50
UNROLL = 10

_INF = float("inf")
_EPS = float(2.0 ** -18)


def _rtne_bf16(x):
    b = lax.bitcast_convert_type(x, jnp.int32)
    rounded = b + jnp.int32(0x7FFF) + ((b >> 16) & jnp.int32(1))
    return lax.bitcast_convert_type(rounded & jnp.int32(-65536), jnp.float32)


def _make_body(nw, nc, qpw, nchunk, ckeys, ktop):
    citer = ckeys // L
    prefix_v = min(PREFIX_V, citer)

    def body(q_hbm, k_hbm, idx_hbm, dist_hbm,
             qx_v, qy_v, qz_v, kx_v, ky_v, kz_v, ksq_v, bufa_v, bufb_v,
             topk_v, topd_v, topi_v):
        wid = lax.axis_index("s") * nc + lax.axis_index("c")
        iota = lax.iota(jnp.int32, L)
        qbase = wid * (3 * qpw)

        pltpu.sync_copy(q_hbm.at[pl.ds(qbase + 0 * qpw, qpw)], qx_v)
        pltpu.sync_copy(q_hbm.at[pl.ds(qbase + 1 * qpw, qpw)], qy_v)
        pltpu.sync_copy(q_hbm.at[pl.ds(qbase + 2 * qpw, qpw)], qz_v)

        for c in range(nchunk):
            first = c == 0
            last = c == nchunk - 1
            kbase = c * (3 * ckeys)
            pltpu.sync_copy(k_hbm.at[pl.ds(kbase + 0 * ckeys, ckeys)], kx_v)
            pltpu.sync_copy(k_hbm.at[pl.ds(kbase + 1 * ckeys, ckeys)], ky_v)
            pltpu.sync_copy(k_hbm.at[pl.ds(kbase + 2 * ckeys, ckeys)], kz_v)
            base = jnp.int32(c * ckeys)

            def prep(j, _):
                off = j * L
                kx = kx_v[pl.ds(off, L)]
                ky = ky_v[pl.ds(off, L)]
                kz = kz_v[pl.ds(off, L)]
                ksq_v[pl.ds(off, L)] = (kx * kx + ky * ky) + kz * kz
                kx_v[pl.ds(off, L)] = _rtne_bf16(kx)
                ky_v[pl.ds(off, L)] = _rtne_bf16(ky)
                kz_v[pl.ds(off, L)] = _rtne_bf16(kz)
                return _

            lax.fori_loop(0, citer, prep, 0)

            def pair_body(g, _, first=first, last=last, base=base):
                def qconst(qi):
                    qs = jnp.zeros((L,), jnp.int32) + qi
                    qx = plsc.load_gather(qx_v, [qs])
                    qy = plsc.load_gather(qy_v, [qs])
                    qz = plsc.load_gather(qz_v, [qs])
                    qsq = (qx * qx + qy * qy) + qz * qz
                    qbx = _rtne_bf16(qx)
                    qby = _rtne_bf16(qy)
                    qbz = _rtne_bf16(qz)
                    return (qbx + qbx, qby + qby, qbz + qbz, qsq)

                q0 = g * 2
                q1 = q0 + 1
                qc0 = qconst(q0)
                qc1 = qconst(q1)
                soff0 = q0 * ktop
                soff1 = q1 * ktop

                def dist(qc, kxb, kyb, kzb, ksq):
                    q2x, q2y, q2z, qsq = qc
                    dot2 = (q2x * kxb + q2y * kyb) + q2z * kzb
                    return (qsq - dot2) + ksq

                def merge_exact(car, ci, qc, nvalid=None):
                    top_k, top_i = car
                    il = ci - base
                    il = jnp.minimum(jnp.maximum(il, 0), ckeys - 1)
                    kxb = plsc.load_gather(kx_v, [il])
                    kyb = plsc.load_gather(ky_v, [il])
                    kzb = plsc.load_gather(kz_v, [il])
                    ksq = plsc.load_gather(ksq_v, [il])
                    d2 = dist(qc, kxb, kyb, kzb, ksq)
                    d2c = jnp.maximum(d2, jnp.float32(0.0))
                    zk = (jnp.asarray(ci, jnp.float32) * jnp.float32(_EPS)
                          - jnp.float32(2.0))
                    key = jnp.where(d2c > jnp.float32(0.0), d2c, zk)
                    if nvalid is not None:
                        key = jnp.where(iota < nvalid, key,
                                        jnp.float32(_INF))
                    sk, si = plsc.sort_key_val(key, ci)
                    rk = lax.rev(sk, (0,))
                    ri = lax.rev(si, (0,))
                    keep = top_k <= rk
                    lo_k = jnp.minimum(top_k, rk)
                    lo_i = jnp.where(keep, top_i, ri)
                    return tuple(plsc.sort_key_val(lo_k, lo_i))

                if first:
                    top0 = (jnp.full((L,), _INF, jnp.float32),
                            jnp.zeros((L,), jnp.int32))
                    top1 = top0

                    def pre(j, car):
                        t0, t1 = car
                        ci = base + j * L + iota
                        return (merge_exact(t0, ci, qc0),
                                merge_exact(t1, ci, qc1))

                    top0, top1 = lax.fori_loop(
                        0, prefix_v, pre, (top0, top1))
                    scan_lo = prefix_v
                else:
                    top0 = (topk_v[pl.ds(soff0, L)], topi_v[pl.ds(soff0, L)])
                    top1 = (topk_v[pl.ds(soff1, L)], topi_v[pl.ds(soff1, L)])
                    scan_lo = 0

                def thrv(top):
                    return jnp.broadcast_to(
                        jnp.maximum(jnp.max(top[0]), jnp.float32(0.0)), (L,))

                thr0 = thrv(top0)
                thr1 = thrv(top1)
                civ0 = base + jnp.int32(scan_lo * L) + iota
                ones = jnp.full((L,), 1, jnp.int32)

                def scan(i, car):
                    offa, offb, civ = car
                    off16 = i * L
                    kxb = kx_v[pl.ds(off16, L)]
                    kyb = ky_v[pl.ds(off16, L)]
                    kzb = kz_v[pl.ds(off16, L)]
                    ksq = ksq_v[pl.ds(off16, L)]
                    m0 = dist(qc0, kxb, kyb, kzb, ksq) <= thr0
                    cs0 = plsc.cumsum(ones, mask=m0)
                    plsc.store_scatter(bufa_v, [offa + cs0], civ, mask=m0)
                    offa = offa + plsc.all_reduce_population_count(m0)
                    m1 = dist(qc1, kxb, kyb, kzb, ksq) <= thr1
                    cs1 = plsc.cumsum(ones, mask=m1)
                    plsc.store_scatter(bufb_v, [offb + cs1], civ, mask=m1)
                    offb = offb + plsc.all_reduce_population_count(m1)
                    return offa, offb, civ + jnp.int32(L)

                minus1 = jnp.full((L,), -1, jnp.int32)
                offa, offb, _c = plsc.parallel_loop(
                    scan_lo, citer, 1, unroll=UNROLL,
                    carry=(minus1, minus1, civ0))(scan)

                def drain(top, off_s, buf_v, qc):
                    cnt = jnp.max(off_s) + 1
                    nfull = lax.shift_right_logical(cnt, 4)
                    ntail = cnt & jnp.int32(15)

                    def mb(j, car):
                        civ = buf_v[pl.ds(j * L, L)]
                        return merge_exact(car, civ, qc)

                    top = lax.fori_loop(0, nfull, mb, top)
                    civ_t = buf_v[pl.ds(nfull * L, L)]
                    return merge_exact(top, civ_t, qc, nvalid=ntail)

                top0 = drain(top0, offa, bufa_v, qc0)
                top1 = drain(top1, offb, bufb_v, qc1)

                for soff, top in ((soff0, top0), (soff1, top1)):
                    if last:
                        topd_v[pl.ds(soff, L)] = jnp.maximum(
                            top[0], jnp.float32(0.0))
                    else:
                        topk_v[pl.ds(soff, L)] = top[0]
                    topi_v[pl.ds(soff, L)] = top[1]
                return _

            lax.fori_loop(0, qpw // 2, pair_body, 0)

        obase = wid * (qpw * ktop)
        pltpu.sync_copy(topi_v, idx_hbm.at[pl.ds(obase, qpw * ktop)])
        pltpu.sync_copy(topd_v, dist_hbm.at[pl.ds(obase, qpw * ktop)])

    return body


def _make_call(nw, nc, nq, nkey, nchunk, ktop, interpret=False):
    qpw = nq // nw
    ckeys = nkey // nchunk
    mesh = plsc.VectorSubcoreMesh(core_axis_name="c", subcore_axis_name="s",
                                  num_cores=nc, num_subcores=nw // nc)
    f = pl.kernel(
        _make_body(nw, nc, qpw, nchunk, ckeys, ktop),
        out_type=(
            jax.ShapeDtypeStruct((nq * ktop,), jnp.int32),
            jax.ShapeDtypeStruct((nq * ktop,), jnp.float32),
        ),
        mesh=mesh,
        compiler_params=pltpu.CompilerParams(
            needs_layout_passes=False,
            use_tc_tiling_on_sc=False,
        ),
        scratch_types=[
            pltpu.VMEM((qpw,), jnp.float32),
            pltpu.VMEM((qpw,), jnp.float32),
            pltpu.VMEM((qpw,), jnp.float32),
            pltpu.VMEM((ckeys,), jnp.float32),
            pltpu.VMEM((ckeys,), jnp.float32),
            pltpu.VMEM((ckeys,), jnp.float32),
            pltpu.VMEM((ckeys,), jnp.float32),
            pltpu.VMEM((ckeys + L,), jnp.int32),
            pltpu.VMEM((ckeys + L,), jnp.int32),
            pltpu.VMEM((qpw * ktop,), jnp.float32),
            pltpu.VMEM((qpw * ktop,), jnp.float32),
            pltpu.VMEM((qpw * ktop,), jnp.int32),
        ],
        interpret=interpret,
    )

    def call(queries, keys):
        q_flat = queries.T.reshape(3, nw, qpw).transpose(1, 0, 2).reshape(-1)
        k_flat = keys.T.reshape(3, nchunk, ckeys).transpose(1, 0, 2).reshape(-1)
        idx_flat, d2_flat = f(q_flat, k_flat)
        knn_idx = idx_flat.reshape(nq, ktop)
        knn_dist = jnp.sqrt(d2_flat.reshape(nq, ktop))
        return knn_idx, knn_dist

    return call


@jax.jit
def _knn(queries, keys):
    return _make_call(NW, NC, NQ, NKEY, NCHUNK, KTOP)(queries, keys)


def kernel(queries, keys, k):
    return _knn(queries, keys)


# speedup vs baseline: 1.3347x; 1.3347x over previous
import jax
import jax.numpy as jnp
from jax import lax
from jax.experimental import pallas as pl
from jax.experimental.pallas import tpu as pltpu
from jax.experimental.pallas import tpu_sc as plsc

L = 16
NW = 32
NC = 2
NQ = 4096
NKEY = 100000
NCHUNK = 5
KTOP = 16
PREFIX_V = 50
UNROLL = 5

_INF = float("inf")
_EPS = float(2.0 ** -18)


def _rtne_bf16(x):
    b = lax.bitcast_convert_type(x, jnp.int32)
    rounded = b + jnp.int32(0x7FFF) + ((b >> 16) & jnp.int32(1))
    return lax.bitcast_convert_type(rounded & jnp.int32(-65536), jnp.float32)


def _make_body(nw, nc, qpw, nchunk, ckeys, ktop):
    citer = ckeys // L
    prefix_v = min(PREFIX_V, citer)

    def body(q_hbm, k_hbm, idx_hbm, dist_hbm,
             qx_v, qy_v, qz_v, kx_v, ky_v, kz_v, ksq_v, bufa_v, bufb_v,
             topk_v, topd_v, topi_v):
        wid = lax.axis_index("s") * nc + lax.axis_index("c")
        iota = lax.iota(jnp.int32, L)
        qbase = wid * (3 * qpw)

        pltpu.sync_copy(q_hbm.at[pl.ds(qbase + 0 * qpw, qpw)], qx_v)
        pltpu.sync_copy(q_hbm.at[pl.ds(qbase + 1 * qpw, qpw)], qy_v)
        pltpu.sync_copy(q_hbm.at[pl.ds(qbase + 2 * qpw, qpw)], qz_v)

        for c in range(nchunk):
            first = c == 0
            last = c == nchunk - 1
            kbase = c * (3 * ckeys)
            pltpu.sync_copy(k_hbm.at[pl.ds(kbase + 0 * ckeys, ckeys)], kx_v)
            pltpu.sync_copy(k_hbm.at[pl.ds(kbase + 1 * ckeys, ckeys)], ky_v)
            pltpu.sync_copy(k_hbm.at[pl.ds(kbase + 2 * ckeys, ckeys)], kz_v)
            base = jnp.int32(c * ckeys)

            def prep(j, _):
                off = j * L
                kx = kx_v[pl.ds(off, L)]
                ky = ky_v[pl.ds(off, L)]
                kz = kz_v[pl.ds(off, L)]
                ksq_v[pl.ds(off, L)] = (kx * kx + ky * ky) + kz * kz
                kx_v[pl.ds(off, L)] = _rtne_bf16(kx)
                ky_v[pl.ds(off, L)] = _rtne_bf16(ky)
                kz_v[pl.ds(off, L)] = _rtne_bf16(kz)
                return _

            lax.fori_loop(0, citer, prep, 0)

            def pair_body(g, _, first=first, last=last, base=base):
                def qconst(qi):
                    qs = jnp.zeros((L,), jnp.int32) + qi
                    qx = plsc.load_gather(qx_v, [qs])
                    qy = plsc.load_gather(qy_v, [qs])
                    qz = plsc.load_gather(qz_v, [qs])
                    qsq = (qx * qx + qy * qy) + qz * qz
                    qbx = _rtne_bf16(qx)
                    qby = _rtne_bf16(qy)
                    qbz = _rtne_bf16(qz)
                    return (qbx + qbx, qby + qby, qbz + qbz, qsq)

                q0 = g * 2
                q1 = q0 + 1
                qc0 = qconst(q0)
                qc1 = qconst(q1)
                soff0 = q0 * ktop
                soff1 = q1 * ktop

                def dist(qc, kxb, kyb, kzb, ksq):
                    q2x, q2y, q2z, qsq = qc
                    dot2 = (q2x * kxb + q2y * kyb) + q2z * kzb
                    return (qsq - dot2) + ksq

                def merge_exact(car, ci, qc, nvalid=None):
                    top_k, top_i = car
                    il = ci - base
                    il = jnp.minimum(jnp.maximum(il, 0), ckeys - 1)
                    kxb = plsc.load_gather(kx_v, [il])
                    kyb = plsc.load_gather(ky_v, [il])
                    kzb = plsc.load_gather(kz_v, [il])
                    ksq = plsc.load_gather(ksq_v, [il])
                    d2 = dist(qc, kxb, kyb, kzb, ksq)
                    d2c = jnp.maximum(d2, jnp.float32(0.0))
                    zk = (jnp.asarray(ci, jnp.float32) * jnp.float32(_EPS)
                          - jnp.float32(2.0))
                    key = jnp.where(d2c > jnp.float32(0.0), d2c, zk)
                    if nvalid is not None:
                        key = jnp.where(iota < nvalid, key,
                                        jnp.float32(_INF))
                    sk, si = plsc.sort_key_val(key, ci)
                    rk = lax.rev(sk, (0,))
                    ri = lax.rev(si, (0,))
                    keep = top_k <= rk
                    lo_k = jnp.minimum(top_k, rk)
                    lo_i = jnp.where(keep, top_i, ri)
                    return tuple(plsc.sort_key_val(lo_k, lo_i))

                if first:
                    top0 = (jnp.full((L,), _INF, jnp.float32),
                            jnp.zeros((L,), jnp.int32))
                    top1 = top0

                    def pre(j, car):
                        t0, t1 = car
                        ci = base + j * L + iota
                        return (merge_exact(t0, ci, qc0),
                                merge_exact(t1, ci, qc1))

                    top0, top1 = lax.fori_loop(
                        0, prefix_v, pre, (top0, top1))
                    scan_lo = prefix_v
                else:
                    top0 = (topk_v[pl.ds(soff0, L)], topi_v[pl.ds(soff0, L)])
                    top1 = (topk_v[pl.ds(soff1, L)], topi_v[pl.ds(soff1, L)])
                    scan_lo = 0

                def thrv(top):
                    return jnp.broadcast_to(
                        jnp.maximum(jnp.max(top[0]), jnp.float32(0.0)), (L,))

                thr0 = thrv(top0)
                thr1 = thrv(top1)
                civ0 = base + jnp.int32(scan_lo * L) + iota
                ones = jnp.full((L,), 1, jnp.int32)

                def scan(i, car):
                    offa, offb, civ = car
                    off16 = i * L
                    kxb = kx_v[pl.ds(off16, L)]
                    kyb = ky_v[pl.ds(off16, L)]
                    kzb = kz_v[pl.ds(off16, L)]
                    ksq = ksq_v[pl.ds(off16, L)]
                    m0 = dist(qc0, kxb, kyb, kzb, ksq) <= thr0
                    cs0 = plsc.cumsum(ones, mask=m0)
                    plsc.store_scatter(bufa_v, [offa + cs0], civ, mask=m0)
                    offa = offa + plsc.all_reduce_population_count(m0)
                    m1 = dist(qc1, kxb, kyb, kzb, ksq) <= thr1
                    cs1 = plsc.cumsum(ones, mask=m1)
                    plsc.store_scatter(bufb_v, [offb + cs1], civ, mask=m1)
                    offb = offb + plsc.all_reduce_population_count(m1)
                    return offa, offb, civ + jnp.int32(L)

                minus1 = jnp.full((L,), -1, jnp.int32)
                offa, offb, _c = plsc.parallel_loop(
                    scan_lo, citer, 1, unroll=UNROLL,
                    carry=(minus1, minus1, civ0))(scan)

                def drain(top, off_s, buf_v, qc):
                    cnt = jnp.max(off_s) + 1
                    nfull = lax.shift_right_logical(cnt, 4)
                    ntail = cnt & jnp.int32(15)

                    def mb(j, car):
                        civ = buf_v[pl.ds(j * L, L)]
                        return merge_exact(car, civ, qc)

                    top = lax.fori_loop(0, nfull, mb, top)
                    civ_t = buf_v[pl.ds(nfull * L, L)]
                    return merge_exact(top, civ_t, qc, nvalid=ntail)

                top0 = drain(top0, offa, bufa_v, qc0)
                top1 = drain(top1, offb, bufb_v, qc1)

                for soff, top in ((soff0, top0), (soff1, top1)):
                    if last:
                        topd_v[pl.ds(soff, L)] = jnp.maximum(
                            top[0], jnp.float32(0.0))
                    else:
                        topk_v[pl.ds(soff, L)] = top[0]
                    topi_v[pl.ds(soff, L)] = top[1]
                return _

            lax.fori_loop(0, qpw // 2, pair_body, 0)

        obase = wid * (qpw * ktop)
        pltpu.sync_copy(topi_v, idx_hbm.at[pl.ds(obase, qpw * ktop)])
        pltpu.sync_copy(topd_v, dist_hbm.at[pl.ds(obase, qpw * ktop)])

    return body


def _make_call(nw, nc, nq, nkey, nchunk, ktop, interpret=False):
    qpw = nq // nw
    ckeys = nkey // nchunk
    mesh = plsc.VectorSubcoreMesh(core_axis_name="c", subcore_axis_name="s",
                                  num_cores=nc, num_subcores=nw // nc)
    f = pl.kernel(
        _make_body(nw, nc, qpw, nchunk, ckeys, ktop),
        out_type=(
            jax.ShapeDtypeStruct((nq * ktop,), jnp.int32),
            jax.ShapeDtypeStruct((nq * ktop,), jnp.float32),
        ),
        mesh=mesh,
        compiler_params=pltpu.CompilerParams(
            needs_layout_passes=False,
            use_tc_tiling_on_sc=False,
        ),
        scratch_types=[
            pltpu.VMEM((qpw,), jnp.float32),
            pltpu.VMEM((qpw,), jnp.float32),
            pltpu.VMEM((qpw,), jnp.float32),
            pltpu.VMEM((ckeys,), jnp.float32),
            pltpu.VMEM((ckeys,), jnp.float32),
            pltpu.VMEM((ckeys,), jnp.float32),
            pltpu.VMEM((ckeys,), jnp.float32),
            pltpu.VMEM((ckeys + L,), jnp.int32),
            pltpu.VMEM((ckeys + L,), jnp.int32),
            pltpu.VMEM((qpw * ktop,), jnp.float32),
            pltpu.VMEM((qpw * ktop,), jnp.float32),
            pltpu.VMEM((qpw * ktop,), jnp.int32),
        ],
        interpret=interpret,
    )

    def call(queries, keys):
        q_flat = queries.T.reshape(3, nw, qpw).transpose(1, 0, 2).reshape(-1)
        k_flat = keys.T.reshape(3, nchunk, ckeys).transpose(1, 0, 2).reshape(-1)
        idx_flat, d2_flat = f(q_flat, k_flat)
        knn_idx = idx_flat.reshape(nq, ktop)
        knn_dist = jnp.sqrt(d2_flat.reshape(nq, ktop))
        return knn_idx, knn_dist

    return call


@jax.jit
def _knn(queries, keys):
    return _make_call(NW, NC, NQ, NKEY, NCHUNK, KTOP)(queries, keys)


def kernel(queries, keys, k):
    return _knn(queries, keys)
